# baseline (speedup 1.0000x reference)
import functools

import jax
import jax.numpy as jnp
from jax import lax
from jax.experimental import pallas as pl
from jax.experimental.pallas import tpu as pltpu
from jax.experimental.pallas import tpu_sc as plsc

_BETA = 0.9
_LAM = 3.0
_EPS = 1e-4


def _loss_body(p_ref, t_ref, g_ref, o_ref):
    p = p_ref[...]
    g = g_ref[...]
    t = t_ref[0, 0, :]
    m = jnp.max(p, axis=1, keepdims=True)
    e = jnp.exp(p - m)
    z = jnp.sum(e, axis=1, keepdims=True)
    y = jnp.clip(e / z, _EPS, 1.0 - _EPS)
    s1 = jnp.sum(y, axis=1)
    s2 = jnp.sum(y * y, axis=1)
    d = jnp.sum(g * y, axis=1)
    cls = lax.broadcasted_iota(jnp.int32, p.shape, 1)
    pt = jnp.sum(jnp.where(cls == t[:, None], p, 0.0), axis=1)
    ce = m[:, 0] + jnp.log(z[:, 0]) - pt
    elr = jnp.log(1.0 - (_BETA * d + (1.0 - _BETA) * s2 / s1))
    o_ref[0, 0, :] = ce + _LAM * elr


def _fused_loss(predictions, targets, gathered, block_rows=512,
                interpret=False):
    B, C = predictions.shape
    nb = B // block_rows
    t3 = targets.reshape(nb, 1, block_rows)
    out = pl.pallas_call(
        _loss_body,
        grid=(nb,),
        in_specs=[
            pl.BlockSpec((block_rows, C), lambda i: (i, 0)),
            pl.BlockSpec((1, 1, block_rows), lambda i: (i, 0, 0)),
            pl.BlockSpec((block_rows, C), lambda i: (i, 0)),
        ],
        out_specs=pl.BlockSpec((1, 1, block_rows), lambda i: (i, 0, 0)),
        out_shape=jax.ShapeDtypeStruct((nb, 1, block_rows), jnp.float32),
        interpret=interpret,
    )(predictions, t3, gathered)
    return out.reshape(B)


def _sc_gather(table, indices, chunk=32):
    V, D = table.shape
    B = indices.shape[0]
    info = plsc.get_sparse_core_info()
    nw = info.num_cores * info.num_subcores
    b_per_w = B // nw
    n_ch = b_per_w // chunk
    idx3 = indices.reshape(nw, n_ch, chunk)
    mesh = plsc.VectorSubcoreMesh(core_axis_name="c", subcore_axis_name="s")

    @functools.partial(
        pl.kernel, mesh=mesh,
        out_type=jax.ShapeDtypeStruct((B, D), jnp.float32),
        compiler_params=pltpu.CompilerParams(use_tc_tiling_on_sc=False),
        scratch_types=[
            pltpu.VMEM((n_ch, chunk), jnp.int32),
            pltpu.VMEM((chunk, D), jnp.float32),
            pltpu.VMEM((chunk, D), jnp.float32),
            pltpu.SemaphoreType.DMA,
            pltpu.SemaphoreType.DMA,
            pltpu.SemaphoreType.DMA,
        ],
    )
    def k(table_hbm, idx_hbm, out_hbm, idx_v, rows_a, rows_b, sem_a,
          sem_b, sem_out):
        wid = lax.axis_index("s") * info.num_cores + lax.axis_index("c")
        base = wid * b_per_w
        pltpu.sync_copy(idx_hbm.at[wid], idx_v)
        bufs = (rows_a, rows_b)
        sems = (sem_a, sem_b)
        pltpu.make_async_copy(table_hbm.at[idx_v.at[0]], rows_a, sem_a
                              ).start()
        for ci in range(n_ch):
            cur, nxt = bufs[ci % 2], bufs[(ci + 1) % 2]
            pltpu.make_async_copy(table_hbm.at[idx_v.at[ci]], cur,
                                  sems[ci % 2]).wait()
            if ci + 1 < n_ch:
                pltpu.make_async_copy(table_hbm.at[idx_v.at[ci + 1]], nxt,
                                      sems[(ci + 1) % 2]).start()
            out_cp = pltpu.make_async_copy(
                cur, out_hbm.at[pl.ds(base + ci * chunk, chunk)], sem_out)
            out_cp.start()
            out_cp.wait()

    return k(table, idx3)


def kernel(predictions, targets, indices, targets_buffer):
    gathered = _sc_gather(targets_buffer, indices)
    y = jnp.clip(jax.nn.softmax(predictions, axis=1), _EPS, 1.0 - _EPS)
    s1 = jnp.sum(y, axis=1)
    s2 = jnp.sum(y * y, axis=1)
    d = jnp.sum(gathered * y, axis=1)
    lp = jax.nn.log_softmax(predictions, axis=1)
    ce = -jnp.take_along_axis(lp, targets[:, None], axis=1)[:, 0]
    return ce + _LAM * jnp.log(1.0 - (_BETA * d + (1.0 - _BETA) * s2 / s1))


# speedup vs baseline: 3.0110x; 3.0110x over previous
import jax
import jax.numpy as jnp
from jax import lax
from jax.experimental import pallas as pl
from jax.experimental.pallas import tpu as pltpu

_BETA = 0.9
_LAM = 3.0
_EPS = 1e-4
_J = 64


def _body(idx_ref, p_ref, t_ref, tb_ref, o_ref, g_buf, sem):
    i = pl.program_id(0)
    nb = pl.num_programs(0)

    def issue(step, slot):
        for j in range(_J):
            r = idx_ref[step * _J + j]
            pltpu.make_async_copy(
                tb_ref.at[r], g_buf.at[slot, j], sem.at[slot]).start()

    @pl.when(i == 0)
    def _():
        issue(i, 0)

    @pl.when(i + 1 < nb)
    def _():
        issue(i + 1, (i + 1) % 2)

    slot = i % 2
    for j in range(_J):
        pltpu.make_async_copy(
            tb_ref.at[0], g_buf.at[slot, j], sem.at[slot]).wait()

    p = p_ref[...]
    t = t_ref[0, 0, :]
    g = g_buf[slot]
    m = jnp.max(p, axis=1, keepdims=True)
    e = jnp.exp(p - m)
    z = jnp.sum(e, axis=1, keepdims=True)
    y = jnp.clip(e / z, _EPS, 1.0 - _EPS)
    s1 = jnp.sum(y, axis=1)
    s2 = jnp.sum(y * y, axis=1)
    d = jnp.sum(g * y, axis=1)
    cls = lax.broadcasted_iota(jnp.int32, p.shape, 1)
    pt = jnp.sum(jnp.where(cls == t[:, None], p, 0.0), axis=1)
    ce = m[:, 0] + jnp.log(z[:, 0]) - pt
    elr = jnp.log(1.0 - (_BETA * d + (1.0 - _BETA) * s2 / s1))
    o_ref[0, 0, :] = ce + _LAM * elr


def kernel(predictions, targets, indices, targets_buffer):
    B, C = predictions.shape
    nb = B // _J
    t3 = targets.reshape(nb, 1, _J)

    grid_spec = pltpu.PrefetchScalarGridSpec(
        num_scalar_prefetch=1,
        grid=(nb,),
        in_specs=[
            pl.BlockSpec((_J, C), lambda i, idx: (i, 0)),
            pl.BlockSpec((1, 1, _J), lambda i, idx: (i, 0, 0)),
            pl.BlockSpec(memory_space=pl.ANY),
        ],
        out_specs=pl.BlockSpec((1, 1, _J), lambda i, idx: (i, 0, 0)),
        scratch_shapes=[
            pltpu.VMEM((2, _J, C), jnp.float32),
            pltpu.SemaphoreType.DMA((2,)),
        ],
    )
    out = pl.pallas_call(
        _body,
        grid_spec=grid_spec,
        out_shape=jax.ShapeDtypeStruct((nb, 1, _J), jnp.float32),
    )(indices, predictions, t3, targets_buffer)
    return out.reshape(B)


# speedup vs baseline: 3.0182x; 1.0024x over previous
import jax
import jax.numpy as jnp
from jax import lax
from jax.experimental import pallas as pl
from jax.experimental.pallas import tpu as pltpu

_BETA = 0.9
_LAM = 3.0
_EPS = 1e-4
_J = 64


def _body(idx_ref, p_ref, t_ref, tb_ref, o_ref, g_buf, sem):
    i = pl.program_id(0)
    nb = pl.num_programs(0)

    def issue(step, slot):
        for j in range(_J):
            r = idx_ref[step * _J + j]
            pltpu.make_async_copy(
                tb_ref.at[r], g_buf.at[slot, j], sem.at[slot]).start()

    @pl.when(i == 0)
    def _():
        issue(i, 0)

    @pl.when(i + 1 < nb)
    def _():
        issue(i + 1, (i + 1) % 2)

    slot = i % 2
    p = p_ref[...]
    t = t_ref[0, 0, :]
    m = jnp.max(p, axis=1, keepdims=True)
    e = jnp.exp(p - m)
    z = jnp.sum(e, axis=1, keepdims=True)
    y = jnp.clip(e / z, _EPS, 1.0 - _EPS)
    s1 = jnp.sum(y, axis=1)
    s2 = jnp.sum(y * y, axis=1)
    cls = lax.broadcasted_iota(jnp.int32, p.shape, 1)
    pt = jnp.sum(jnp.where(cls == t[:, None], p, 0.0), axis=1)
    ce = m[:, 0] + jnp.log(z[:, 0]) - pt

    for j in range(_J):
        pltpu.make_async_copy(
            tb_ref.at[0], g_buf.at[slot, j], sem.at[slot]).wait()
    g = g_buf[slot]
    d = jnp.sum(g * y, axis=1)
    elr = jnp.log(1.0 - (_BETA * d + (1.0 - _BETA) * s2 / s1))
    o_ref[0, 0, :] = ce + _LAM * elr


def kernel(predictions, targets, indices, targets_buffer):
    B, C = predictions.shape
    nb = B // _J
    t3 = targets.reshape(nb, 1, _J)

    grid_spec = pltpu.PrefetchScalarGridSpec(
        num_scalar_prefetch=1,
        grid=(nb,),
        in_specs=[
            pl.BlockSpec((_J, C), lambda i, idx: (i, 0)),
            pl.BlockSpec((1, 1, _J), lambda i, idx: (i, 0, 0)),
            pl.BlockSpec(memory_space=pl.ANY),
        ],
        out_specs=pl.BlockSpec((1, 1, _J), lambda i, idx: (i, 0, 0)),
        scratch_shapes=[
            pltpu.VMEM((2, _J, C), jnp.float32),
            pltpu.SemaphoreType.DMA((2,)),
        ],
    )
    out = pl.pallas_call(
        _body,
        grid_spec=grid_spec,
        out_shape=jax.ShapeDtypeStruct((nb, 1, _J), jnp.float32),
    )(indices, predictions, t3, targets_buffer)
    return out.reshape(B)
